# baseline (speedup 1.0000x reference)
import functools

import jax
import jax.numpy as jnp
from jax import lax
from jax.experimental import pallas as pl
from jax.experimental.pallas import tpu as pltpu
from jax.experimental.pallas import tpu_sc as plsc

NU = 25000
NV = 50000
D = 64
H = 32
E = 800000
B = 16384
NLAYERS = 3
NLEV = NLAYERS + 1

NC = 2
NS = 16
C = 128
EPT = E // NS
NB = 4
EPTP = 50048
NCHUNK = EPTP // C
RPT = NV // NS
ZR = 125
QPT = B // NS
QC = 16

XROWS = NC * NLEV * NV


def _body(x0, e3, ui, ii, preds, xt,
          ebuf, gidx_v, sidx_v, rows_v, rows_o,
          uq, iq, pv,
          acc,
          sem_e, sem_g, sem_s, sem_q):
    c = lax.axis_index("c")
    s = lax.axis_index("s")
    base_c = c * (NLEV * NV)
    i16 = lax.iota(jnp.int32, 16)
    z16 = jnp.zeros((16,), jnp.float32)

    for k in range(RPT // ZR):
        off = c * NV + s * RPT + k * ZR
        pltpu.sync_copy(x0.at[pl.ds(off, ZR)], rows_v[0].at[pl.ds(0, ZR)])
        pltpu.sync_copy(rows_v[0].at[pl.ds(0, ZR)],
                        xt.at[pl.ds(base_c + s * RPT + k * ZR, ZR)])

    def edge_loads(j, b):
        jj = jnp.minimum(j, NCHUNK - 1)
        pltpu.async_copy(e3.at[s, jj], ebuf[b], sem_e[b])

    def wait_edge(b):
        pltpu.make_async_copy(e3.at[0, 0], ebuf[b], sem_e[b]).wait()

    def start_gather(b, src_off):
        for g in range(C // 16):
            gidx_v[b][pl.ds(g * 16, 16)] = ebuf[b][0, pl.ds(g * 16, 16)] + src_off
        pltpu.async_copy(xt.at[gidx_v[b]], rows_v[b], sem_g[b])

    def wait_gather(b):
        pltpu.make_async_copy(xt.at[gidx_v[b]], rows_v[b], sem_g[b]).wait()

    def scale_rows(b, bo):
        for g in range(C // 16):
            sidx_v[bo][pl.ds(g * 16, 16)] = ebuf[b][1, pl.ds(g * 16, 16)]

        def _e(eb, _):
            wv = plsc.bitcast(ebuf[b][2, pl.ds(eb * 16, 16)], jnp.float32)
            for i in range(16):
                e = eb * 16 + i
                ws = wv[i]
                rows_o[bo][e, pl.ds(0, 16)] = rows_v[b][e, pl.ds(0, 16)] * ws
                rows_o[bo][e, pl.ds(16, 16)] = rows_v[b][e, pl.ds(16, 16)] * ws
            return 0

        lax.fori_loop(0, C // 16, _e, 0)

    def start_scatter(bo):
        pltpu.async_copy(rows_o[bo], acc.at[sidx_v[bo]], sem_s[bo], add=True)

    def wait_scatter(bo):
        pltpu.make_async_copy(rows_o[bo], acc.at[sidx_v[bo]], sem_s[bo]).wait()

    def layer(l, _):
        src_off = base_c + l * NV

        def _zb(r, _2):
            rows_v[0][r, pl.ds(0, 16)] = z16
            rows_v[0][r, pl.ds(16, 16)] = z16
            return 0

        lax.fori_loop(0, ZR, _zb, 0)
        for k in range(RPT // ZR):
            pltpu.async_copy(rows_v[0].at[pl.ds(0, ZR)],
                             acc.at[pl.ds(s * RPT + k * ZR, ZR)], sem_g[0])
        for k in range(RPT // ZR):
            pltpu.make_async_copy(rows_v[0].at[pl.ds(0, ZR)],
                                  acc.at[pl.ds(s * RPT, ZR)], sem_g[0]).wait()
        plsc.subcore_barrier()

        for b in range(NB):
            edge_loads(jnp.int32(b), b)
        for b in range(NB - 1):
            wait_edge(b)
            start_gather(b, src_off)
        for j0 in range(NB):
            b = j0
            bo = j0 % 2
            bg = (j0 + 3) % 4
            wait_gather(b)
            if j0 >= 2:
                wait_scatter(bo)
            scale_rows(b, bo)
            start_scatter(bo)
            wait_edge(bg)
            start_gather(bg, src_off)
            edge_loads(jnp.int32(j0 + 4), b)

        def quad(t, _2):
            j = 4 + 4 * t
            for r in range(4):
                b = r
                bo = r % 2
                bg = (r + 3) % 4
                jj = j + r
                wait_gather(b)
                wait_scatter(bo)
                scale_rows(b, bo)
                start_scatter(bo)
                wait_edge(bg)
                start_gather(bg, src_off)
                edge_loads(jj + 4, b)
            return 0

        lax.fori_loop(0, (NCHUNK - 7) // 4, quad, 0)

        for (jn, b) in ((388, 0), (389, 1), (390, 2)):
            bo = jn % 2
            wait_gather(b)
            wait_scatter(bo)
            scale_rows(b, bo)
            start_scatter(bo)
        wait_edge(3)
        wait_scatter(1)
        wait_scatter(0)
        plsc.subcore_barrier()

        dst_off = base_c + (l + 1) * NV + s * RPT
        pltpu.sync_copy(acc.at[pl.ds(s * RPT, RPT)], xt.at[pl.ds(dst_off, RPT)])
        return 0

    lax.fori_loop(0, NLAYERS, layer, 0)
    plsc.subcore_barrier()

    pltpu.sync_copy(ui.at[pl.ds(s * QPT, QPT)], uq)
    pltpu.sync_copy(ii.at[pl.ds(s * QPT, QPT)], iq)

    def qidx_build(k, bq):
        kk = jnp.minimum(k, QPT // QC - 1)
        sl = pl.ds(kk * QC, 16)
        uqs = uq[sl]
        iqs = iq[sl]
        for l in range(NLEV):
            gidx_v[bq][pl.ds(l * 16, 16)] = uqs + (base_c + l * NV)
            gidx_v[bq][pl.ds(64 + l * 16, 16)] = iqs + (base_c + l * NV + NU)
        pltpu.async_copy(xt.at[gidx_v[bq]], rows_v[bq], sem_q[bq])

    def qwait(bq):
        pltpu.make_async_copy(xt.at[gidx_v[bq]], rows_v[bq], sem_q[bq]).wait()

    def qcompute(k, bq):
        qwait(bq)

        def _dot(f, acc_v):
            cf = jnp.full((16,), f, jnp.int32)
            u = ((plsc.load_gather(rows_v[bq], [i16, cf])
                  + plsc.load_gather(rows_v[bq], [i16 + 16, cf]))
                 + (plsc.load_gather(rows_v[bq], [i16 + 32, cf])
                    + plsc.load_gather(rows_v[bq], [i16 + 48, cf])))
            v = ((plsc.load_gather(rows_v[bq], [i16 + 64, cf])
                  + plsc.load_gather(rows_v[bq], [i16 + 80, cf]))
                 + (plsc.load_gather(rows_v[bq], [i16 + 96, cf])
                    + plsc.load_gather(rows_v[bq], [i16 + 112, cf])))
            return acc_v + u * v

        accv = lax.fori_loop(0, H, _dot, z16)
        pv[pl.ds(k * QC, 16)] = accv * jnp.float32(1.0 / (NLEV * NLEV))

    qidx_build(jnp.int32(0), 0)

    def qpair(t, _):
        k = 2 * t
        qidx_build(k + 1, 1)
        qcompute(k, 0)
        qidx_build(k + 2, 0)
        qcompute(k + 1, 1)
        return 0

    lax.fori_loop(0, QPT // QC // 2, qpair, 0)
    qwait(0)
    pltpu.sync_copy(pv, preds.at[pl.ds(c * B + s * QPT, QPT)])


@functools.partial(jax.jit, static_argnums=())
def _lightgcn_sc(x0, e3, ui, ii):
    mesh = plsc.VectorSubcoreMesh(
        core_axis_name="c", subcore_axis_name="s", num_cores=NC, num_subcores=NS)
    scratch = (
        [pltpu.VMEM((3, C), jnp.int32) for _ in range(NB)]
        + [pltpu.VMEM((C,), jnp.int32) for _ in range(NB)]
        + [pltpu.VMEM((C,), jnp.int32) for _ in range(2)]
        + [pltpu.VMEM((C, H), jnp.float32) for _ in range(NB)]
        + [pltpu.VMEM((C, H), jnp.float32) for _ in range(2)]
        + [pltpu.VMEM((QPT,), jnp.int32) for _ in range(2)]
        + [pltpu.VMEM((QPT,), jnp.float32)]
        + [pltpu.VMEM_SHARED((NV, H), jnp.float32)]
        + [pltpu.SemaphoreType.DMA for _ in range(2 * NB + 4)]
    )

    def body(x0r, e3r, uir, iir, predsr, xtr, *sc):
        ebuf, gidx_v = sc[0:4], sc[4:8]
        sidx_v = sc[8:10]
        rows_v, rows_o = sc[10:14], sc[14:16]
        uq, iq, pv = sc[16], sc[17], sc[18]
        acc = sc[19]
        sems = sc[20:]
        sem_e, sem_g = sems[0:4], sems[4:8]
        sem_s, sem_q = sems[8:10], sems[10:12]
        _body(x0r, e3r, uir, iir, predsr, xtr,
              ebuf, gidx_v, sidx_v, rows_v, rows_o,
              uq, iq, pv, acc,
              sem_e, sem_g, sem_s, sem_q)

    k = pl.kernel(
        body,
        out_type=(
            jax.ShapeDtypeStruct((NC * B,), jnp.float32),
            jax.ShapeDtypeStruct((XROWS, H), jnp.float32),
        ),
        mesh=mesh,
        scratch_types=scratch,
        compiler_params=pltpu.CompilerParams(
            use_tc_tiling_on_sc=False, needs_layout_passes=False),
    )
    preds_part, _ = k(x0, e3, ui, ii)
    return preds_part


def kernel(user_table, item_table, edge_weight, edge_index, user_input, item_input):
    x0 = jnp.concatenate(
        [user_table[:, :H], item_table[:, :H],
         user_table[:, H:], item_table[:, H:]], axis=0)
    pad = ((0, 0), (0, EPTP - EPT))
    esp = jnp.pad(edge_index[0].reshape(NS, EPT), pad)
    edp = jnp.pad(edge_index[1].reshape(NS, EPT), pad)
    ewp = jnp.pad(lax.bitcast_convert_type(edge_weight, jnp.int32).reshape(NS, EPT), pad)
    e3 = jnp.stack([esp.reshape(NS, NCHUNK, C), edp.reshape(NS, NCHUNK, C),
                    ewp.reshape(NS, NCHUNK, C)], axis=2)
    part = _lightgcn_sc(x0, e3, user_input, item_input)
    return part[:B] + part[B:]


# speedup vs baseline: 1.0206x; 1.0206x over previous
import functools

import jax
import jax.numpy as jnp
from jax import lax
from jax.experimental import pallas as pl
from jax.experimental.pallas import tpu as pltpu
from jax.experimental.pallas import tpu_sc as plsc

NU = 25000
NV = 50000
D = 64
H = 32
E = 800000
B = 16384
NLAYERS = 3
NLEV = NLAYERS + 1

NC = 2
NS = 16
C = 128
EPT = E // NS
NB = 4
EPTP = 50048
NCHUNK = EPTP // C
RPT = NV // NS
ZR = 125
QPT = B // NS
QC = 16

XROWS = NC * NLEV * NV


def _body(x0, e3, ui, ii, preds, xt,
          ebuf, gidx_v, sidx_v, rows_v, rows_o,
          uq, iq, pv,
          acc,
          sem_e, sem_g, sem_s, sem_q):
    c = lax.axis_index("c")
    s = lax.axis_index("s")
    base_c = c * (NLEV * NV)
    i16 = lax.iota(jnp.int32, 16)
    z16 = jnp.zeros((16,), jnp.float32)

    for k in range(RPT // ZR):
        off = c * NV + s * RPT + k * ZR
        pltpu.sync_copy(x0.at[pl.ds(off, ZR)], rows_v[0].at[pl.ds(0, ZR)])
        pltpu.sync_copy(rows_v[0].at[pl.ds(0, ZR)],
                        xt.at[pl.ds(base_c + s * RPT + k * ZR, ZR)])

    def edge_loads(j, b):
        off = jnp.minimum(j, NCHUNK - 1) * C
        pltpu.async_copy(e3.at[s, :, pl.ds(off, C)], ebuf[b], sem_e[b])

    def wait_edge(b):
        pltpu.make_async_copy(e3.at[0, :, pl.ds(0, C)], ebuf[b], sem_e[b]).wait()

    def start_gather(b, src_off):
        for g in range(C // 16):
            gidx_v[b][pl.ds(g * 16, 16)] = ebuf[b][0, pl.ds(g * 16, 16)] + src_off
        pltpu.async_copy(xt.at[gidx_v[b]], rows_v[b], sem_g[b])

    def wait_gather(b):
        pltpu.make_async_copy(xt.at[gidx_v[b]], rows_v[b], sem_g[b]).wait()

    def scale_rows(b, bo):
        for g in range(C // 16):
            sidx_v[bo][pl.ds(g * 16, 16)] = ebuf[b][1, pl.ds(g * 16, 16)]

        def _e(eb, _):
            wv = plsc.bitcast(ebuf[b][2, pl.ds(eb * 16, 16)], jnp.float32)
            for i in range(16):
                e = eb * 16 + i
                ws = wv[i]
                rows_o[bo][e, pl.ds(0, 16)] = rows_v[b][e, pl.ds(0, 16)] * ws
                rows_o[bo][e, pl.ds(16, 16)] = rows_v[b][e, pl.ds(16, 16)] * ws
            return 0

        lax.fori_loop(0, C // 16, _e, 0)

    def start_scatter(bo):
        pltpu.async_copy(rows_o[bo], acc.at[sidx_v[bo]], sem_s[bo], add=True)

    def wait_scatter(bo):
        pltpu.make_async_copy(rows_o[bo], acc.at[sidx_v[bo]], sem_s[bo]).wait()

    def layer(l, _):
        src_off = base_c + l * NV

        def _zb(r, _2):
            rows_v[0][r, pl.ds(0, 16)] = z16
            rows_v[0][r, pl.ds(16, 16)] = z16
            return 0

        lax.fori_loop(0, ZR, _zb, 0)
        for k in range(RPT // ZR):
            pltpu.async_copy(rows_v[0].at[pl.ds(0, ZR)],
                             acc.at[pl.ds(s * RPT + k * ZR, ZR)], sem_g[0])
        for k in range(RPT // ZR):
            pltpu.make_async_copy(rows_v[0].at[pl.ds(0, ZR)],
                                  acc.at[pl.ds(s * RPT, ZR)], sem_g[0]).wait()
        plsc.subcore_barrier()

        for b in range(NB):
            edge_loads(jnp.int32(b), b)
        for b in range(NB - 1):
            wait_edge(b)
            start_gather(b, src_off)
        for j0 in range(NB):
            b = j0
            bo = j0 % 2
            bg = (j0 + 3) % 4
            wait_gather(b)
            if j0 >= 2:
                wait_scatter(bo)
            scale_rows(b, bo)
            start_scatter(bo)
            wait_edge(bg)
            start_gather(bg, src_off)
            edge_loads(jnp.int32(j0 + 4), b)

        def quad(t, _2):
            j = 4 + 4 * t
            for r in range(4):
                b = r
                bo = r % 2
                bg = (r + 3) % 4
                jj = j + r
                wait_gather(b)
                wait_scatter(bo)
                scale_rows(b, bo)
                start_scatter(bo)
                wait_edge(bg)
                start_gather(bg, src_off)
                edge_loads(jj + 4, b)
            return 0

        lax.fori_loop(0, (NCHUNK - 7) // 4, quad, 0)

        for (jn, b) in ((388, 0), (389, 1), (390, 2)):
            bo = jn % 2
            wait_gather(b)
            wait_scatter(bo)
            scale_rows(b, bo)
            start_scatter(bo)
        wait_edge(3)
        wait_scatter(1)
        wait_scatter(0)
        plsc.subcore_barrier()

        dst_off = base_c + (l + 1) * NV + s * RPT
        pltpu.sync_copy(acc.at[pl.ds(s * RPT, RPT)], xt.at[pl.ds(dst_off, RPT)])
        return 0

    lax.fori_loop(0, NLAYERS, layer, 0)
    plsc.subcore_barrier()

    pltpu.sync_copy(ui.at[pl.ds(s * QPT, QPT)], uq)
    pltpu.sync_copy(ii.at[pl.ds(s * QPT, QPT)], iq)

    def qidx_build(k, bq):
        kk = jnp.minimum(k, QPT // QC - 1)
        sl = pl.ds(kk * QC, 16)
        uqs = uq[sl]
        iqs = iq[sl]
        for l in range(NLEV):
            gidx_v[bq][pl.ds(l * 16, 16)] = uqs + (base_c + l * NV)
            gidx_v[bq][pl.ds(64 + l * 16, 16)] = iqs + (base_c + l * NV + NU)
        pltpu.async_copy(xt.at[gidx_v[bq]], rows_v[bq], sem_q[bq])

    def qwait(bq):
        pltpu.make_async_copy(xt.at[gidx_v[bq]], rows_v[bq], sem_q[bq]).wait()

    def qcompute(k, bq):
        qwait(bq)

        def _dot(f, acc_v):
            cf = jnp.full((16,), f, jnp.int32)
            u = ((plsc.load_gather(rows_v[bq], [i16, cf])
                  + plsc.load_gather(rows_v[bq], [i16 + 16, cf]))
                 + (plsc.load_gather(rows_v[bq], [i16 + 32, cf])
                    + plsc.load_gather(rows_v[bq], [i16 + 48, cf])))
            v = ((plsc.load_gather(rows_v[bq], [i16 + 64, cf])
                  + plsc.load_gather(rows_v[bq], [i16 + 80, cf]))
                 + (plsc.load_gather(rows_v[bq], [i16 + 96, cf])
                    + plsc.load_gather(rows_v[bq], [i16 + 112, cf])))
            return acc_v + u * v

        accv = lax.fori_loop(0, H, _dot, z16)
        pv[pl.ds(k * QC, 16)] = accv * jnp.float32(1.0 / (NLEV * NLEV))

    qidx_build(jnp.int32(0), 0)

    def qpair(t, _):
        k = 2 * t
        qidx_build(k + 1, 1)
        qcompute(k, 0)
        qidx_build(k + 2, 0)
        qcompute(k + 1, 1)
        return 0

    lax.fori_loop(0, QPT // QC // 2, qpair, 0)
    qwait(0)
    pltpu.sync_copy(pv, preds.at[pl.ds(c * B + s * QPT, QPT)])


@functools.partial(jax.jit, static_argnums=())
def _lightgcn_sc(x0, e3, ui, ii):
    mesh = plsc.VectorSubcoreMesh(
        core_axis_name="c", subcore_axis_name="s", num_cores=NC, num_subcores=NS)
    scratch = (
        [pltpu.VMEM((3, C), jnp.int32) for _ in range(NB)]
        + [pltpu.VMEM((C,), jnp.int32) for _ in range(NB)]
        + [pltpu.VMEM((C,), jnp.int32) for _ in range(2)]
        + [pltpu.VMEM((C, H), jnp.float32) for _ in range(NB)]
        + [pltpu.VMEM((C, H), jnp.float32) for _ in range(2)]
        + [pltpu.VMEM((QPT,), jnp.int32) for _ in range(2)]
        + [pltpu.VMEM((QPT,), jnp.float32)]
        + [pltpu.VMEM_SHARED((NV, H), jnp.float32)]
        + [pltpu.SemaphoreType.DMA for _ in range(2 * NB + 4)]
    )

    def body(x0r, e3r, uir, iir, predsr, xtr, *sc):
        ebuf, gidx_v = sc[0:4], sc[4:8]
        sidx_v = sc[8:10]
        rows_v, rows_o = sc[10:14], sc[14:16]
        uq, iq, pv = sc[16], sc[17], sc[18]
        acc = sc[19]
        sems = sc[20:]
        sem_e, sem_g = sems[0:4], sems[4:8]
        sem_s, sem_q = sems[8:10], sems[10:12]
        _body(x0r, e3r, uir, iir, predsr, xtr,
              ebuf, gidx_v, sidx_v, rows_v, rows_o,
              uq, iq, pv, acc,
              sem_e, sem_g, sem_s, sem_q)

    k = pl.kernel(
        body,
        out_type=(
            jax.ShapeDtypeStruct((NC * B,), jnp.float32),
            jax.ShapeDtypeStruct((XROWS, H), jnp.float32),
        ),
        mesh=mesh,
        scratch_types=scratch,
        compiler_params=pltpu.CompilerParams(
            use_tc_tiling_on_sc=False, needs_layout_passes=False),
    )
    preds_part, _ = k(x0, e3, ui, ii)
    return preds_part


def kernel(user_table, item_table, edge_weight, edge_index, user_input, item_input):
    x0 = jnp.concatenate(
        [user_table[:, :H], item_table[:, :H],
         user_table[:, H:], item_table[:, H:]], axis=0)
    pad = ((0, 0), (0, EPTP - EPT))
    esp = jnp.pad(edge_index[0].reshape(NS, EPT), pad)
    edp = jnp.pad(edge_index[1].reshape(NS, EPT), pad)
    ewp = jnp.pad(lax.bitcast_convert_type(edge_weight, jnp.int32).reshape(NS, EPT), pad)
    e3 = jnp.stack([esp, edp, ewp], axis=1)
    part = _lightgcn_sc(x0, e3, user_input, item_input)
    return part[:B] + part[B:]


# speedup vs baseline: 1.0475x; 1.0263x over previous
import functools

import jax
import jax.numpy as jnp
from jax import lax
from jax.experimental import pallas as pl
from jax.experimental.pallas import tpu as pltpu
from jax.experimental.pallas import tpu_sc as plsc

NU = 25000
NV = 50000
D = 64
H = 32
E = 800000
B = 16384
NLAYERS = 3
NLEV = NLAYERS + 1

NC = 2
NS = 16
C = 128
EPT = E // NS
NB = 4
EPTP = 50048
NCHUNK = EPTP // C
RPT = NV // NS
ZR = 125
QPT = B // NS
QC = 16

XROWS = NC * NLEV * NV


def _body(x0, ew, es, ed, ui, ii, preds, xt,
          src_v, dst_v, w_v, gidx_v, sidx_v, rows_v, rows_o,
          uq, iq, pv,
          acc,
          sem_e, sem_g, sem_s, sem_q):
    c = lax.axis_index("c")
    s = lax.axis_index("s")
    base_c = c * (NLEV * NV)
    i16 = lax.iota(jnp.int32, 16)
    z16 = jnp.zeros((16,), jnp.float32)

    for k in range(RPT // ZR):
        off = c * NV + s * RPT + k * ZR
        pltpu.sync_copy(x0.at[pl.ds(off, ZR)], rows_v[0].at[pl.ds(0, ZR)])
        pltpu.sync_copy(rows_v[0].at[pl.ds(0, ZR)],
                        xt.at[pl.ds(base_c + s * RPT + k * ZR, ZR)])

    def edge_loads(j, b):
        off = jnp.minimum(j, NCHUNK - 1) * C
        pltpu.async_copy(es.at[s, pl.ds(off, C)], src_v[b], sem_e[b])
        pltpu.async_copy(ed.at[s, pl.ds(off, C)], dst_v[b], sem_e[b])
        pltpu.async_copy(ew.at[s, pl.ds(off, C)], w_v[b], sem_e[b])

    def wait_edge(b):
        pltpu.make_async_copy(es.at[0, pl.ds(0, C)], src_v[b], sem_e[b]).wait()
        pltpu.make_async_copy(ed.at[0, pl.ds(0, C)], dst_v[b], sem_e[b]).wait()
        pltpu.make_async_copy(ew.at[0, pl.ds(0, C)], w_v[b], sem_e[b]).wait()

    def start_gather(b, src_off):
        for g in range(C // 16):
            gidx_v[b][pl.ds(g * 16, 16)] = src_v[b][pl.ds(g * 16, 16)] + src_off
        pltpu.async_copy(xt.at[gidx_v[b]], rows_v[b], sem_g[b])

    def wait_gather(b):
        pltpu.make_async_copy(xt.at[gidx_v[b]], rows_v[b], sem_g[b]).wait()

    def scale_rows(b, bo):
        for g in range(C // 16):
            sidx_v[bo][pl.ds(g * 16, 16)] = dst_v[b][pl.ds(g * 16, 16)]

        def _e(eb, _):
            wv = w_v[b][pl.ds(eb * 16, 16)]
            for i in range(16):
                e = eb * 16 + i
                ws = wv[i]
                rows_o[bo][e, pl.ds(0, 16)] = rows_v[b][e, pl.ds(0, 16)] * ws
                rows_o[bo][e, pl.ds(16, 16)] = rows_v[b][e, pl.ds(16, 16)] * ws
            return 0

        lax.fori_loop(0, C // 16, _e, 0)

    def start_scatter(bo):
        pltpu.async_copy(rows_o[bo], acc.at[sidx_v[bo]], sem_s[bo], add=True)

    def wait_scatter(bo):
        pltpu.make_async_copy(rows_o[bo], acc.at[sidx_v[bo]], sem_s[bo]).wait()

    def layer(l, _):
        src_off = base_c + l * NV

        def _zb(r, _2):
            rows_v[0][r, pl.ds(0, 16)] = z16
            rows_v[0][r, pl.ds(16, 16)] = z16
            return 0

        lax.fori_loop(0, ZR, _zb, 0)
        for k in range(RPT // ZR):
            pltpu.async_copy(rows_v[0].at[pl.ds(0, ZR)],
                             acc.at[pl.ds(s * RPT + k * ZR, ZR)], sem_g[0])
        for k in range(RPT // ZR):
            pltpu.make_async_copy(rows_v[0].at[pl.ds(0, ZR)],
                                  acc.at[pl.ds(s * RPT, ZR)], sem_g[0]).wait()
        plsc.subcore_barrier()

        for b in range(NB):
            edge_loads(jnp.int32(b), b)
        for b in range(NB - 1):
            wait_edge(b)
            start_gather(b, src_off)
        for j0 in range(NB):
            b = j0
            bo = j0 % 2
            bg = (j0 + 3) % 4
            wait_gather(b)
            if j0 >= 2:
                wait_scatter(bo)
            scale_rows(b, bo)
            start_scatter(bo)
            wait_edge(bg)
            start_gather(bg, src_off)
            edge_loads(jnp.int32(j0 + 4), b)

        def quad(t, _2):
            j = 4 + 4 * t
            for r in range(4):
                b = r
                bo = r % 2
                bg = (r + 3) % 4
                jj = j + r
                wait_gather(b)
                wait_scatter(bo)
                scale_rows(b, bo)
                start_scatter(bo)
                wait_edge(bg)
                start_gather(bg, src_off)
                edge_loads(jj + 4, b)
            return 0

        lax.fori_loop(0, (NCHUNK - 7) // 4, quad, 0)

        for (jn, b) in ((388, 0), (389, 1), (390, 2)):
            bo = jn % 2
            wait_gather(b)
            wait_scatter(bo)
            scale_rows(b, bo)
            start_scatter(bo)
        wait_edge(3)
        wait_scatter(1)
        wait_scatter(0)
        plsc.subcore_barrier()

        dst_off = base_c + (l + 1) * NV + s * RPT
        pltpu.sync_copy(acc.at[pl.ds(s * RPT, RPT)], xt.at[pl.ds(dst_off, RPT)])
        return 0

    lax.fori_loop(0, NLAYERS, layer, 0)
    plsc.subcore_barrier()

    pltpu.sync_copy(ui.at[pl.ds(s * QPT, QPT)], uq)
    pltpu.sync_copy(ii.at[pl.ds(s * QPT, QPT)], iq)

    def qidx_build(k, bq):
        kk = jnp.minimum(k, QPT // QC - 1)
        sl = pl.ds(kk * QC, 16)
        uqs = uq[sl]
        iqs = iq[sl]
        for l in range(NLEV):
            gidx_v[bq][pl.ds(l * 16, 16)] = uqs + (base_c + l * NV)
            gidx_v[bq][pl.ds(64 + l * 16, 16)] = iqs + (base_c + l * NV + NU)
        pltpu.async_copy(xt.at[gidx_v[bq]], rows_v[bq], sem_q[bq])

    def qwait(bq):
        pltpu.make_async_copy(xt.at[gidx_v[bq]], rows_v[bq], sem_q[bq]).wait()

    def qcompute(k, bq):
        qwait(bq)

        def _dot(f, acc_v):
            cf = jnp.full((16,), f, jnp.int32)
            u = ((plsc.load_gather(rows_v[bq], [i16, cf])
                  + plsc.load_gather(rows_v[bq], [i16 + 16, cf]))
                 + (plsc.load_gather(rows_v[bq], [i16 + 32, cf])
                    + plsc.load_gather(rows_v[bq], [i16 + 48, cf])))
            v = ((plsc.load_gather(rows_v[bq], [i16 + 64, cf])
                  + plsc.load_gather(rows_v[bq], [i16 + 80, cf]))
                 + (plsc.load_gather(rows_v[bq], [i16 + 96, cf])
                    + plsc.load_gather(rows_v[bq], [i16 + 112, cf])))
            return acc_v + u * v

        accv = lax.fori_loop(0, H, _dot, z16)
        pv[pl.ds(k * QC, 16)] = accv * jnp.float32(1.0 / (NLEV * NLEV))

    qidx_build(jnp.int32(0), 0)

    def qpair(t, _):
        k = 2 * t
        qidx_build(k + 1, 1)
        qcompute(k, 0)
        qidx_build(k + 2, 0)
        qcompute(k + 1, 1)
        return 0

    lax.fori_loop(0, QPT // QC // 2, qpair, 0)
    qwait(0)
    pltpu.sync_copy(pv, preds.at[pl.ds(c * B + s * QPT, QPT)])


@functools.partial(jax.jit, static_argnums=())
def _lightgcn_sc(x0, ewp, esp, edp, ui, ii):
    mesh = plsc.VectorSubcoreMesh(
        core_axis_name="c", subcore_axis_name="s", num_cores=NC, num_subcores=NS)
    scratch = (
        [pltpu.VMEM((C,), jnp.int32) for _ in range(NB)]
        + [pltpu.VMEM((C,), jnp.int32) for _ in range(NB)]
        + [pltpu.VMEM((C,), jnp.float32) for _ in range(NB)]
        + [pltpu.VMEM((C,), jnp.int32) for _ in range(NB)]
        + [pltpu.VMEM((C,), jnp.int32) for _ in range(2)]
        + [pltpu.VMEM((C, H), jnp.float32) for _ in range(NB)]
        + [pltpu.VMEM((C, H), jnp.float32) for _ in range(2)]
        + [pltpu.VMEM((QPT,), jnp.int32) for _ in range(2)]
        + [pltpu.VMEM((QPT,), jnp.float32)]
        + [pltpu.VMEM_SHARED((NV, H), jnp.float32)]
        + [pltpu.SemaphoreType.DMA for _ in range(2 * NB + 4)]
    )

    def body(x0r, ewr, esr, edr, uir, iir, predsr, xtr, *sc):
        src_v, dst_v, w_v, gidx_v = sc[0:4], sc[4:8], sc[8:12], sc[12:16]
        sidx_v = sc[16:18]
        rows_v, rows_o = sc[18:22], sc[22:24]
        uq, iq, pv = sc[24], sc[25], sc[26]
        acc = sc[27]
        sems = sc[28:]
        sem_e, sem_g = sems[0:4], sems[4:8]
        sem_s, sem_q = sems[8:10], sems[10:12]
        _body(x0r, ewr, esr, edr, uir, iir, predsr, xtr,
              src_v, dst_v, w_v, gidx_v, sidx_v, rows_v, rows_o,
              uq, iq, pv, acc,
              sem_e, sem_g, sem_s, sem_q)

    k = pl.kernel(
        body,
        out_type=(
            jax.ShapeDtypeStruct((NC * B,), jnp.float32),
            jax.ShapeDtypeStruct((XROWS, H), jnp.float32),
        ),
        mesh=mesh,
        scratch_types=scratch,
        compiler_params=pltpu.CompilerParams(
            use_tc_tiling_on_sc=False, needs_layout_passes=False),
    )
    preds_part, _ = k(x0, ewp, esp, edp, ui, ii)
    return preds_part


def kernel(user_table, item_table, edge_weight, edge_index, user_input, item_input):
    x0 = jnp.concatenate(
        [user_table[:, :H], item_table[:, :H],
         user_table[:, H:], item_table[:, H:]], axis=0)
    pad = ((0, 0), (0, EPTP - EPT))
    esp = jnp.pad(edge_index[0].reshape(NS, EPT), pad)
    edp = jnp.pad(edge_index[1].reshape(NS, EPT), pad)
    ewp = jnp.pad(edge_weight.reshape(NS, EPT), pad)
    part = _lightgcn_sc(x0, ewp, esp, edp, user_input, item_input)
    return part[:B] + part[B:]


# speedup vs baseline: 1.0913x; 1.0418x over previous
import functools

import jax
import jax.numpy as jnp
from jax import lax
from jax.experimental import pallas as pl
from jax.experimental.pallas import tpu as pltpu
from jax.experimental.pallas import tpu_sc as plsc

NU = 25000
NV = 50000
D = 64
H = 32
E = 800000
B = 16384
NLAYERS = 3
NLEV = NLAYERS + 1

NC = 2
NS = 16
C = 128
EPT = E // NS
NB = 4
EPTP = 50048
NCHUNK = EPTP // C
RPT = NV // NS
ZR = 125
QPT = B // NS
QC = 16

XROWS = NC * NLEV * NV


def _body(x0, ew, es, ed, ui, ii, preds, xt,
          src_v, dst_v, w_v, gidx_v, sidx_v, rows_v, rows_o,
          uq, iq, pv,
          acc,
          sem_e, sem_g, sem_s, sem_q):
    c = lax.axis_index("c")
    s = lax.axis_index("s")
    base_c = c * (NLEV * NV)
    i16 = lax.iota(jnp.int32, 16)
    z16 = jnp.zeros((16,), jnp.float32)

    for k in range(RPT // ZR):
        off = c * NV + s * RPT + k * ZR
        pltpu.sync_copy(x0.at[pl.ds(off, ZR)], rows_v[0].at[pl.ds(0, ZR)])
        pltpu.sync_copy(rows_v[0].at[pl.ds(0, ZR)],
                        xt.at[pl.ds(base_c + s * RPT + k * ZR, ZR)])

    def edge_loads(j, b):
        off = jnp.minimum(j, NCHUNK - 1) * C
        pltpu.async_copy(es.at[s, pl.ds(off, C)], src_v[b], sem_e[b])
        pltpu.async_copy(ed.at[s, pl.ds(off, C)], dst_v[b], sem_e[b])
        pltpu.async_copy(ew.at[s, pl.ds(off, C)], w_v[b], sem_e[b])

    def wait_edge(b):
        pltpu.make_async_copy(es.at[0, pl.ds(0, C)], src_v[b], sem_e[b]).wait()
        pltpu.make_async_copy(ed.at[0, pl.ds(0, C)], dst_v[b], sem_e[b]).wait()
        pltpu.make_async_copy(ew.at[0, pl.ds(0, C)], w_v[b], sem_e[b]).wait()

    def start_gather(b, src_off):
        for g in range(C // 16):
            gidx_v[b][pl.ds(g * 16, 16)] = src_v[b][pl.ds(g * 16, 16)] + src_off
        pltpu.async_copy(xt.at[gidx_v[b]], rows_v[b], sem_g[b])

    def wait_gather(b):
        pltpu.make_async_copy(xt.at[gidx_v[b]], rows_v[b], sem_g[b]).wait()

    def scale_rows(b, bo):
        for g in range(C // 16):
            sidx_v[bo][pl.ds(g * 16, 16)] = dst_v[b][pl.ds(g * 16, 16)]

        def _e(eb, _):
            wv = w_v[b][pl.ds(eb * 16, 16)]
            for i in range(16):
                e = eb * 16 + i
                ws = wv[i]
                rows_o[bo][e, pl.ds(0, 16)] = rows_v[b][e, pl.ds(0, 16)] * ws
                rows_o[bo][e, pl.ds(16, 16)] = rows_v[b][e, pl.ds(16, 16)] * ws
            return 0

        lax.fori_loop(0, C // 16, _e, 0)

    def start_scatter(bo):
        pltpu.async_copy(rows_o[bo], acc.at[sidx_v[bo]], sem_s[bo], add=True)

    def wait_scatter(bo):
        pltpu.make_async_copy(rows_o[bo], acc.at[sidx_v[bo]], sem_s[bo]).wait()

    def layer(l, _):
        src_off = base_c + l * NV

        def _zb(r, _2):
            rows_v[0][r, pl.ds(0, 16)] = z16
            rows_v[0][r, pl.ds(16, 16)] = z16
            return 0

        lax.fori_loop(0, ZR, _zb, 0)
        for k in range(RPT // ZR):
            pltpu.async_copy(rows_v[0].at[pl.ds(0, ZR)],
                             acc.at[pl.ds(s * RPT + k * ZR, ZR)], sem_g[0])
        for k in range(RPT // ZR):
            pltpu.make_async_copy(rows_v[0].at[pl.ds(0, ZR)],
                                  acc.at[pl.ds(s * RPT, ZR)], sem_g[0]).wait()
        plsc.subcore_barrier()

        for b in range(NB):
            edge_loads(jnp.int32(b), b)
        for b in range(NB - 1):
            wait_edge(b)
            start_gather(b, src_off)
        for j0 in range(NB):
            b = j0
            bo = j0 % 2
            bg = (j0 + 3) % 4
            wait_gather(b)
            if j0 >= 2:
                wait_scatter(bo)
            scale_rows(b, bo)
            start_scatter(bo)
            wait_edge(bg)
            start_gather(bg, src_off)
            edge_loads(jnp.int32(j0 + 4), b)

        def quad(t, _2):
            j = 4 + 4 * t
            for r in range(4):
                b = r
                bo = r % 2
                bg = (r + 3) % 4
                jj = j + r
                wait_gather(b)
                wait_scatter(bo)
                scale_rows(b, bo)
                start_scatter(bo)
                wait_edge(bg)
                start_gather(bg, src_off)
                edge_loads(jj + 4, b)
            return 0

        lax.fori_loop(0, (NCHUNK - 7) // 4, quad, 0)

        for (jn, b) in ((388, 0), (389, 1), (390, 2)):
            bo = jn % 2
            wait_gather(b)
            wait_scatter(bo)
            scale_rows(b, bo)
            start_scatter(bo)
        wait_edge(3)
        wait_scatter(1)
        wait_scatter(0)
        plsc.subcore_barrier()

        dst_off = base_c + (l + 1) * NV + s * RPT
        pltpu.sync_copy(acc.at[pl.ds(s * RPT, RPT)], xt.at[pl.ds(dst_off, RPT)])
        return 0

    lax.fori_loop(0, NLAYERS, layer, 0)
    plsc.subcore_barrier()

    pltpu.sync_copy(ui.at[pl.ds(s * QPT, QPT)], uq)
    pltpu.sync_copy(ii.at[pl.ds(s * QPT, QPT)], iq)

    def qidx_build(k, bq):
        kk = jnp.minimum(k, QPT // QC - 1)
        sl = pl.ds(kk * QC, 16)
        uqs = uq[sl]
        iqs = iq[sl]
        for l in range(NLEV):
            gidx_v[bq][pl.ds(l * 16, 16)] = uqs + (base_c + l * NV)
            gidx_v[bq][pl.ds(64 + l * 16, 16)] = iqs + (base_c + l * NV + NU)
        pltpu.async_copy(xt.at[gidx_v[bq]], rows_v[bq], sem_q[bq])

    def qwait(bq):
        pltpu.make_async_copy(xt.at[gidx_v[bq]], rows_v[bq], sem_q[bq]).wait()

    def qcompute(k, bq):
        qwait(bq)

        def _dot(f, acc_v):
            cf = jnp.full((16,), f, jnp.int32)
            u = ((plsc.load_gather(rows_v[bq], [i16, cf])
                  + plsc.load_gather(rows_v[bq], [i16 + 16, cf]))
                 + (plsc.load_gather(rows_v[bq], [i16 + 32, cf])
                    + plsc.load_gather(rows_v[bq], [i16 + 48, cf])))
            v = ((plsc.load_gather(rows_v[bq], [i16 + 64, cf])
                  + plsc.load_gather(rows_v[bq], [i16 + 80, cf]))
                 + (plsc.load_gather(rows_v[bq], [i16 + 96, cf])
                    + plsc.load_gather(rows_v[bq], [i16 + 112, cf])))
            return acc_v + u * v

        accv = lax.fori_loop(0, H, _dot, z16)
        pv[pl.ds(k * QC, 16)] = accv * jnp.float32(1.0 / (NLEV * NLEV))

    qidx_build(jnp.int32(0), 0)

    def qpair(t, _):
        k = 2 * t
        qidx_build(k + 1, 1)
        qcompute(k, 0)
        qidx_build(k + 2, 0)
        qcompute(k + 1, 1)
        return 0

    lax.fori_loop(0, QPT // QC // 2, qpair, 0)
    qwait(0)
    pltpu.sync_copy(pv, preds.at[pl.ds(c * B + s * QPT, QPT)])


@functools.partial(jax.jit, static_argnums=())
def _lightgcn_sc(x0, ewp, esp, edp, ui, ii):
    mesh = plsc.VectorSubcoreMesh(
        core_axis_name="c", subcore_axis_name="s", num_cores=NC, num_subcores=NS)
    scratch = (
        [pltpu.VMEM((C,), jnp.int32) for _ in range(NB)]
        + [pltpu.VMEM((C,), jnp.int32) for _ in range(NB)]
        + [pltpu.VMEM((C,), jnp.float32) for _ in range(NB)]
        + [pltpu.VMEM((C,), jnp.int32) for _ in range(NB)]
        + [pltpu.VMEM((C,), jnp.int32) for _ in range(2)]
        + [pltpu.VMEM((C, H), jnp.float32) for _ in range(NB)]
        + [pltpu.VMEM((C, H), jnp.float32) for _ in range(2)]
        + [pltpu.VMEM((QPT,), jnp.int32) for _ in range(2)]
        + [pltpu.VMEM((QPT,), jnp.float32)]
        + [pltpu.VMEM_SHARED((NV, H), jnp.float32)]
        + [pltpu.SemaphoreType.DMA for _ in range(2 * NB + 4)]
    )

    def body(x0r, ewr, esr, edr, uir, iir, predsr, xtr, *sc):
        src_v, dst_v, w_v, gidx_v = sc[0:4], sc[4:8], sc[8:12], sc[12:16]
        sidx_v = sc[16:18]
        rows_v, rows_o = sc[18:22], sc[22:24]
        uq, iq, pv = sc[24], sc[25], sc[26]
        acc = sc[27]
        sems = sc[28:]
        sem_e, sem_g = sems[0:4], sems[4:8]
        sem_s, sem_q = sems[8:10], sems[10:12]
        _body(x0r, ewr, esr, edr, uir, iir, predsr, xtr,
              src_v, dst_v, w_v, gidx_v, sidx_v, rows_v, rows_o,
              uq, iq, pv, acc,
              sem_e, sem_g, sem_s, sem_q)

    k = pl.kernel(
        body,
        out_type=(
            jax.ShapeDtypeStruct((NC * B,), jnp.float32),
            jax.ShapeDtypeStruct((XROWS, H), jnp.float32),
        ),
        mesh=mesh,
        scratch_types=scratch,
        compiler_params=pltpu.CompilerParams(
            use_tc_tiling_on_sc=False, needs_layout_passes=False),
    )
    preds_part, _ = k(x0, ewp, esp, edp, ui, ii)
    return preds_part


def kernel(user_table, item_table, edge_weight, edge_index, user_input, item_input):
    all_emb = jnp.concatenate([user_table, item_table], axis=0)
    x0 = jnp.concatenate([all_emb[:, :H], all_emb[:, H:]], axis=0)
    pad = ((0, 0), (0, EPTP - EPT))
    esp = jnp.pad(edge_index[0].reshape(NS, EPT), pad)
    edp = jnp.pad(edge_index[1].reshape(NS, EPT), pad)
    ewp = jnp.pad(edge_weight.reshape(NS, EPT), pad)
    part = _lightgcn_sc(x0, ewp, esp, edp, user_input, item_input)
    return part[:B] + part[B:]
